# baseline (speedup 1.0000x reference)
import functools

import jax
import jax.numpy as jnp
from jax.experimental import pallas as pl

N = 2048
D_IN = 128
D_H = 128
D_OUT = 64
BN_EPS = 1e-5


def _dot_kernel(a_ref, b_ref, o_ref):
    o_ref[...] = jnp.dot(a_ref[...].astype(jnp.bfloat16),
                         b_ref[...].astype(jnp.bfloat16),
                         preferred_element_type=jnp.float32)


def _dot_bias_kernel(a_ref, b_ref, bias_ref, o_ref):
    o_ref[...] = jnp.dot(a_ref[...].astype(jnp.bfloat16),
                         b_ref[...].astype(jnp.bfloat16),
                         preferred_element_type=jnp.float32) + bias_ref[...]


def _tdot_bias_relu_kernel(s_ref, adj_ref, b_ref, o_ref):
    s16 = s_ref[...].astype(jnp.bfloat16)
    a16 = adj_ref[...].astype(jnp.bfloat16)
    zT = jax.lax.dot_general(s16, a16, (((0,), (1,)), ((), ())),
                             preferred_element_type=jnp.float32)
    o_ref[...] = jnp.maximum(zT + b_ref[...], 0.0)


def _pdot(a, b):
    return pl.pallas_call(
        _dot_kernel,
        out_shape=jax.ShapeDtypeStruct((a.shape[0], b.shape[1]), jnp.float32),
    )(a, b)


def _pdot_bias(a, b, bias):
    n = b.shape[1]
    return pl.pallas_call(
        _dot_bias_kernel,
        out_shape=jax.ShapeDtypeStruct((a.shape[0], n), jnp.float32),
    )(a, b, bias.reshape(1, n))


def _ptdot_bias_relu(s, adj, b):
    return pl.pallas_call(
        _tdot_bias_relu_kernel,
        out_shape=jax.ShapeDtypeStruct((D_H, N), jnp.float32),
    )(s, adj, b.reshape(D_H, 1))


@functools.partial(jax.jit, static_argnames=())
def kernel(x, adj, W1, b1, W2, b2, W3, b3, bn_gamma, bn_beta, Wout, bout):
    h = x
    for i, (W, b) in enumerate(((W1, b1), (W2, b2), (W3, b3))):
        support = _pdot(h, W)
        actT = _ptdot_bias_relu(support, adj, b)
        mean = jnp.mean(actT, axis=1)
        var = jnp.var(actT, axis=1)
        outT = (actT - mean[:, None]) / jnp.sqrt(var + BN_EPS)[:, None]
        hT = outT * bn_gamma[i][:, None] + bn_beta[i][:, None]
        h = hT.T
    return _pdot_bias(h, Wout, bout)


# speedup vs baseline: 1.0651x; 1.0651x over previous
import functools

import jax
import jax.numpy as jnp
from jax.experimental import pallas as pl

N = 2048
D_IN = 128
D_H = 128
D_OUT = 64
BN_EPS = 1e-5


def _dot_kernel(a_ref, b_ref, o_ref):
    o_ref[...] = jnp.dot(a_ref[...].astype(jnp.bfloat16),
                         b_ref[...].astype(jnp.bfloat16),
                         preferred_element_type=jnp.float32)


def _dot_bias_kernel(a_ref, b_ref, bias_ref, o_ref):
    o_ref[...] = jnp.dot(a_ref[...].astype(jnp.bfloat16),
                         b_ref[...].astype(jnp.bfloat16),
                         preferred_element_type=jnp.float32) + bias_ref[...]


def _tdot_bias_relu_kernel(s_ref, adj_ref, b_ref, o_ref):
    s16 = s_ref[...].astype(jnp.bfloat16)
    a16 = adj_ref[...].astype(jnp.bfloat16)
    zT = jax.lax.dot_general(s16, a16, (((0,), (1,)), ((), ())),
                             preferred_element_type=jnp.float32)
    o_ref[...] = jnp.maximum(zT + b_ref[...], 0.0)


def _pdot(a, b):
    return pl.pallas_call(
        _dot_kernel,
        out_shape=jax.ShapeDtypeStruct((a.shape[0], b.shape[1]), jnp.float32),
    )(a, b)


def _pdot_bias(a, b, bias):
    n = b.shape[1]
    return pl.pallas_call(
        _dot_bias_kernel,
        out_shape=jax.ShapeDtypeStruct((a.shape[0], n), jnp.float32),
    )(a, b, bias.reshape(1, n))


def _ptdot_bias_relu(s, adj, b):
    return pl.pallas_call(
        _tdot_bias_relu_kernel,
        grid=(4,),
        in_specs=[pl.BlockSpec((N, D_H), lambda i: (0, 0)),
                  pl.BlockSpec((N // 4, N), lambda i: (i, 0)),
                  pl.BlockSpec((D_H, 1), lambda i: (0, 0))],
        out_specs=pl.BlockSpec((D_H, N // 4), lambda i: (0, i)),
        out_shape=jax.ShapeDtypeStruct((D_H, N), jnp.float32),
    )(s, adj, b.reshape(D_H, 1))


@functools.partial(jax.jit, static_argnames=())
def kernel(x, adj, W1, b1, W2, b2, W3, b3, bn_gamma, bn_beta, Wout, bout):
    h = x
    for i, (W, b) in enumerate(((W1, b1), (W2, b2), (W3, b3))):
        support = _pdot(h, W)
        actT = _ptdot_bias_relu(support, adj, b)
        mean = jnp.mean(actT, axis=1)
        var = jnp.var(actT, axis=1)
        outT = (actT - mean[:, None]) / jnp.sqrt(var + BN_EPS)[:, None]
        hT = outT * bn_gamma[i][:, None] + bn_beta[i][:, None]
        h = hT.T
    return _pdot_bias(h, Wout, bout)
